# baseline (speedup 1.0000x reference)
import functools

import jax
import jax.numpy as jnp
from jax.experimental import pallas as pl

_N = 16384
_C = 1000
_B = 1024
_M = 10240
_GA = _M // _B
_GB = (_N - _M) // _B


def _body(lab_ref, x_ref, sums_ref, cnt_ref):
    x = x_ref[...]
    lab = lab_ref[...]
    col = jax.lax.broadcasted_iota(jnp.int32, (_B, _C), 1)
    onehot = col == lab
    sumsq1 = jnp.sum(x * x, axis=1, keepdims=True) + 1.0
    a = jnp.sum(jnp.where(onehot, sumsq1 - 2.0 * x, 0.0), axis=0,
                keepdims=True)
    cnt = jnp.sum(jnp.where(onehot, 1.0, 0.0), axis=0, keepdims=True)

    @pl.when(pl.program_id(0) == 0)
    def _():
        sums_ref[...] = jnp.zeros_like(sums_ref)
        cnt_ref[...] = jnp.zeros_like(cnt_ref)

    sums_ref[...] += a * (1.0 / _C)
    cnt_ref[...] += cnt


def _part(x, lab2d, grid, xmap, lmap):
    return pl.pallas_call(
        _body,
        grid=(grid,),
        in_specs=[
            pl.BlockSpec((_B, 1), lmap),
            pl.BlockSpec((_B, _C), xmap),
        ],
        out_specs=[
            pl.BlockSpec((1, _C), lambda i: (0, 0)),
            pl.BlockSpec((1, _C), lambda i: (0, 0)),
        ],
        out_shape=[
            jax.ShapeDtypeStruct((1, _C), jnp.float32),
            jax.ShapeDtypeStruct((1, _C), jnp.float32),
        ],
    )(lab2d, x)


@jax.jit
def kernel(inputs, labels):
    labels2d = labels.astype(jnp.int32).reshape(_N, 1)
    xa = inputs[:_M]
    sb, cb = _part(inputs, labels2d, _GB,
                   lambda i: (i + _GA, 0), lambda i: (i + _GA, 0))
    sa, ca = _part(xa, labels2d, _GA,
                   lambda i: (i, 0), lambda i: (i, 0))
    return ((sa + sb).reshape(_C), (ca + cb).reshape(_C))
